# baseline (speedup 1.0000x reference)
import functools

import jax
import jax.numpy as jnp
from jax import lax
from jax.experimental import pallas as pl
from jax.experimental.pallas import tpu as pltpu
from jax.experimental.pallas import tpu_sc as plsc

_NC = 2
_NS = 16
_L = 16

_K = 80
_NGB = 5
_RBLK = 400


def _gelu(x):
    return 0.5 * x * (1.0 + lax.erf(x * 0.7071067811865476))


def _ln(x, w, b):
    mu = jnp.mean(x, axis=-1, keepdims=True)
    xc = x - mu
    var = jnp.mean(xc * xc, axis=-1, keepdims=True)
    return xc * lax.rsqrt(var + 1e-5) * w + b


def _mlp(x, W1, b1, W2, b2):
    h = _gelu(jnp.dot(x, W1, preferred_element_type=jnp.float32) + b1)
    return jnp.dot(h, W2, preferred_element_type=jnp.float32) + b2


_SB = 400
_CPB = _SB // _K


def _make_sc_counts(N, E, NPAD):
    NW = _NC * _NS
    EW = E // NW
    NB = EW // _SB
    ZB = NPAD // _NS
    mesh = plsc.VectorSubcoreMesh(core_axis_name="c", subcore_axis_name="s")

    @functools.partial(
        pl.kernel,
        out_type=[
            jax.ShapeDtypeStruct((_NC, NPAD), jnp.int32),
            jax.ShapeDtypeStruct((_NC, NPAD), jnp.float32),
        ],
        mesh=mesh,
        compiler_params=pltpu.CompilerParams(needs_layout_passes=False),
        scratch_types=[
            pltpu.VMEM((N,), jnp.float32),
            pltpu.VMEM((_SB,), jnp.int32),
            pltpu.VMEM((_SB,), jnp.int32),
            pltpu.VMEM((_SB,), jnp.int32),
            pltpu.VMEM((_SB,), jnp.int32),
            pltpu.VMEM((_K,), jnp.float32),
            pltpu.VMEM((_K,), jnp.float32),
            pltpu.VMEM((_K,), jnp.int32),
            pltpu.VMEM((_K,), jnp.int32),
            pltpu.VMEM((_K,), jnp.int32),
            pltpu.VMEM((ZB,), jnp.int32),
            pltpu.VMEM((ZB,), jnp.float32),
            pltpu.VMEM_SHARED((NPAD,), jnp.int32),
            pltpu.VMEM_SHARED((NPAD,), jnp.float32),
            pltpu.SemaphoreType.DMA,
            pltpu.SemaphoreType.DMA,
            pltpu.SemaphoreType.DMA,
            pltpu.SemaphoreType.DMA,
        ],
    )
    def sc_counts(ei_h, bd_h, zi_h, zf_h, deg_h, bcnt_h,
                  bd_v, src_a, src_b, dst_a, dst_b, val_a, val_b,
                  dc_a, dc_b, ones_v, zi_v, zf_v, dacc, bacc,
                  ssem_a, ssem_b, asem_a, asem_b):
        c = lax.axis_index("c")
        s = lax.axis_index("s")
        wid = s * _NC + c
        base = wid * EW
        st = [(src_a, dst_a, ssem_a), (src_b, dst_b, ssem_b)]
        ch = [(val_a, dc_a, asem_a), (val_b, dc_b, asem_b)]

        def fire_stage(bidx, p):
            eb = base + bidx * _SB
            pltpu.async_copy(ei_h.at[pl.ds(eb, _SB)], st[p][0], st[p][2])
            pltpu.async_copy(ei_h.at[pl.ds(E + eb, _SB)], st[p][1], st[p][2])

        def wait_stage(p):
            pltpu.make_async_copy(ei_h.at[pl.ds(0, _SB)], st[p][0],
                                  st[p][2]).wait()
            pltpu.make_async_copy(ei_h.at[pl.ds(0, _SB)], st[p][1],
                                  st[p][2]).wait()

        def chunk(p_st, j, p, first):
            sst, dstt, _ = st[p_st]
            val, dc, asem = ch[p]
            if not first:
                pltpu.make_async_copy(ones_v, dacc.at[dc], asem).wait()
                pltpu.make_async_copy(val, bacc.at[dc], asem).wait()
            for k in range(_K // _L):
                sl_in = pl.ds(j * _K + k * _L, _L)
                sl = pl.ds(k * _L, _L)
                val[sl] = plsc.load_gather(bd_v, [sst[sl_in]])
                dc[sl] = dstt[sl_in]
            pltpu.async_copy(ones_v, dacc.at[dc], asem, add=True)
            pltpu.async_copy(val, bacc.at[dc], asem, add=True)

        fire_stage(0, 0)
        pltpu.sync_copy(bd_h, bd_v)
        pltpu.sync_copy(zi_h, zi_v)
        pltpu.sync_copy(zf_h, zf_v)
        pltpu.sync_copy(zi_v, dacc.at[pl.ds(s * ZB, ZB)])
        pltpu.sync_copy(zf_v, bacc.at[pl.ds(s * ZB, ZB)])
        for j in range(_K // _L):
            ones_v[pl.ds(j * _L, _L)] = jnp.ones((_L,), jnp.int32)
        plsc.subcore_barrier()

        wait_stage(0)
        fire_stage(1, 1)
        for j in range(_CPB):
            chunk(0, j, j % 2, first=(j < 2))
        wait_stage(1)
        fire_stage(2, 0)
        for j in range(_CPB):
            chunk(1, j, (j + 1) % 2, first=False)

        def body(g, carry):
            wait_stage(0)
            fire_stage(2 * g + 1, 1)
            for j in range(_CPB):
                chunk(0, j, j % 2, first=False)
            wait_stage(1)
            fire_stage(2 * g + 2, 0)
            for j in range(_CPB):
                chunk(1, j, (j + 1) % 2, first=False)
            return carry

        lax.fori_loop(1, NB // 2, body, 0)

        wait_stage(0)
        for j in range(_CPB):
            chunk(0, j, j % 2, first=False)
        for p in range(2):
            val, dc, asem = ch[p]
            pltpu.make_async_copy(ones_v, dacc.at[dc], asem).wait()
            pltpu.make_async_copy(val, bacc.at[dc], asem).wait()

        plsc.subcore_barrier()
        pltpu.sync_copy(dacc.at[pl.ds(s * ZB, ZB)], deg_h.at[c, pl.ds(s * ZB, ZB)])
        pltpu.sync_copy(bacc.at[pl.ds(s * ZB, ZB)], bcnt_h.at[c, pl.ds(s * ZB, ZB)])

    return sc_counts


def _make_sc_agg(N, E, D, NPAD):
    NW = _NC * _NS
    EW = E // NW
    NB = EW // _SB
    ZR = NPAD // _NS
    RB = 80
    mesh = plsc.VectorSubcoreMesh(core_axis_name="c", subcore_axis_name="s")

    @functools.partial(
        pl.kernel,
        out_type=jax.ShapeDtypeStruct((_NC, NPAD, D), jnp.float32),
        mesh=mesh,
        compiler_params=pltpu.CompilerParams(needs_layout_passes=False),
        scratch_types=[
            pltpu.VMEM((N,), jnp.float32),
            pltpu.VMEM((_SB,), jnp.int32),
            pltpu.VMEM((_SB,), jnp.int32),
            pltpu.VMEM((_SB,), jnp.int32),
            pltpu.VMEM((_SB,), jnp.int32),
            pltpu.VMEM((_K,), jnp.int32),
            pltpu.VMEM((_K,), jnp.int32),
            pltpu.VMEM((_K,), jnp.int32),
            pltpu.VMEM((_K,), jnp.int32),
            pltpu.VMEM((_K, D), jnp.float32),
            pltpu.VMEM((_K, D), jnp.float32),
            pltpu.VMEM_SHARED((NPAD, D), jnp.float32),
            pltpu.SemaphoreType.DMA,
            pltpu.SemaphoreType.DMA,
            pltpu.SemaphoreType.DMA,
            pltpu.SemaphoreType.DMA,
            pltpu.SemaphoreType.DMA,
            pltpu.SemaphoreType.DMA,
        ],
    )
    def sc_agg(ei_h, bd_h, t_h, zrows_h, part_h,
               bd_v, src_a, src_b, dst_a, dst_b, ei_a, ei_b, dc_a, dc_b,
               rows_a, rows_b, acc, ssem_a, ssem_b, gsem_a, gsem_b,
               csem_a, csem_b):
        c = lax.axis_index("c")
        s = lax.axis_index("s")
        wid = s * _NC + c
        base = wid * EW
        st = [(src_a, dst_a, ssem_a), (src_b, dst_b, ssem_b)]
        ch = [(ei_a, dc_a, rows_a, gsem_a, csem_a),
              (ei_b, dc_b, rows_b, gsem_b, csem_b)]

        def fire_stage(bidx, p):
            eb = base + bidx * _SB
            pltpu.async_copy(ei_h.at[pl.ds(eb, _SB)], st[p][0], st[p][2])
            pltpu.async_copy(ei_h.at[pl.ds(E + eb, _SB)], st[p][1], st[p][2])

        def wait_stage(p):
            pltpu.make_async_copy(ei_h.at[pl.ds(0, _SB)], st[p][0],
                                  st[p][2]).wait()
            pltpu.make_async_copy(ei_h.at[pl.ds(0, _SB)], st[p][1],
                                  st[p][2]).wait()

        def chunk(p_st, j, p, first=False, first2=False):
            sst, dstt, _ = st[p_st]
            ei, dc, rows, gsem, csem = ch[p]
            if not (first or first2):
                pltpu.make_async_copy(rows, acc.at[dc], csem).wait()
            for k in range(_K // _L):
                sl_in = pl.ds(j * _K + k * _L, _L)
                sl = pl.ds(k * _L, _L)
                d16 = dstt[sl_in]
                bdd = plsc.load_gather(bd_v, [d16]).astype(jnp.int32)
                ei[sl] = sst[sl_in] + bdd * N
                dc[sl] = d16
            pltpu.async_copy(t_h.at[ei], rows, gsem)
            if not first:
                eo, do, ro, go, co = ch[1 - p]
                pltpu.make_async_copy(t_h.at[eo], ro, go).wait()
                pltpu.async_copy(ro, acc.at[do], co, add=True)

        def final_drain():
            eo, do, ro, go, co = ch[0]
            pltpu.make_async_copy(t_h.at[eo], ro, go).wait()
            pltpu.async_copy(ro, acc.at[do], co, add=True)
            pltpu.make_async_copy(ro, acc.at[do], co).wait()
            e1, d1, r1, g1, c1 = ch[1]
            pltpu.make_async_copy(r1, acc.at[d1], c1).wait()

        fire_stage(0, 0)
        pltpu.sync_copy(bd_h, bd_v)
        pltpu.sync_copy(zrows_h, rows_a)
        for t in range(ZR // RB):
            pltpu.sync_copy(rows_a, acc.at[pl.ds(s * ZR + t * RB, RB)])
        plsc.subcore_barrier()

        wait_stage(0)
        fire_stage(1, 1)
        for j in range(_CPB):
            chunk(0, j, j % 2, first=(j == 0), first2=(j == 1))
        wait_stage(1)
        fire_stage(2, 0)
        for j in range(_CPB):
            chunk(1, j, (j + 1) % 2, first=False)

        def body(g, carry):
            wait_stage(0)
            fire_stage(2 * g + 1, 1)
            for j in range(_CPB):
                chunk(0, j, j % 2, first=False)
            wait_stage(1)
            fire_stage(2 * g + 2, 0)
            for j in range(_CPB):
                chunk(1, j, (j + 1) % 2, first=False)
            return carry

        lax.fori_loop(1, NB // 2, body, 0)

        wait_stage(0)
        for j in range(_CPB):
            chunk(0, j, j % 2)
        final_drain()

        plsc.subcore_barrier()
        for t in range(ZR // RB):
            r0 = s * ZR + t * RB
            pltpu.sync_copy(acc.at[pl.ds(r0, RB)], part_h.at[c, pl.ds(r0, RB)])

    return sc_agg


def _tc1a_body(xt_ref, x0_ref,
               W1r_ref, b1r_ref, W2r_ref, b2r_ref, lnrw_ref, lnrb_ref,
               W1g_ref, b1g_ref, W2g_ref, b2g_ref, lngw_ref, lngb_ref,
               rate_ref, gamma_ref):
    rate_ref[...] = _ln(_mlp(xt_ref[...], W1r_ref[...], b1r_ref[...],
                             W2r_ref[...], b2r_ref[...]),
                        lnrw_ref[...], lnrb_ref[...])
    gamma_ref[...] = _ln(_mlp(x0_ref[...], W1g_ref[...], b1g_ref[...],
                              W2g_ref[...], b2g_ref[...]),
                         lngw_ref[...], lngb_ref[...])


def _tc1b_body(xt_ref, bd_ref, deg_ref, bc_ref, rate_ref,
               T_ref, mfac_ref):
    h = pl.program_id(1)
    bd = bd_ref[...]
    deg = (deg_ref[0] + deg_ref[1]).astype(jnp.float32)
    bc = bc_ref[0] + bc_ref[1]
    p = 1.0 + bd * bc + (1.0 - bd) * (deg - bc)
    rinv = lax.rsqrt(p)
    xt = xt_ref[...]

    @pl.when(h == 0)
    def _():
        mfac_ref[...] = ((1.0 - bd) + bd * rate_ref[...]) * rinv
        T_ref[0] = (2.0 - bd) * rinv * xt

    @pl.when(h == 1)
    def _():
        T_ref[0] = (1.0 - bd) * rinv * xt


def _tc2_body(part_ref, mfac_ref, gamma_ref, W1f_ref, b1f_ref, W2f_ref,
              b2f_ref, out_ref):
    agg = part_ref[0] + part_ref[1]
    z = mfac_ref[...] * agg
    out_ref[...] = _mlp(z, W1f_ref[...], b1f_ref[...], W2f_ref[...],
                        b2f_ref[...]) + gamma_ref[...]


def kernel(xt, x0, edge_index, ind_bd, W1r, b1r, W2r, b2r, lnr_w, lnr_b,
           W1g, b1g, W2g, b2g, lng_w, lng_b, W1f, b1f, W2f, b2f):
    N, D = xt.shape
    E = edge_index.shape[1]
    NPAD = -(-N // 1280) * 1280
    NB = N // _RBLK

    bd_flat = ind_bd.reshape(N)

    zeros_i = jnp.zeros((NPAD // _NS,), jnp.int32)
    zeros_f = jnp.zeros((NPAD // _NS,), jnp.float32)
    ei_flat = edge_index.reshape(2 * E)
    deg_p, bcnt_p = _make_sc_counts(N, E, NPAD)(
        ei_flat, bd_flat, zeros_i, zeros_f)

    row1_spec = pl.BlockSpec((_RBLK, D), lambda i: (i, 0))
    w1_spec = pl.BlockSpec((D, D), lambda i: (0, 0))
    b1_spec = pl.BlockSpec((1, D), lambda i: (0, 0))
    rate, gamma = pl.pallas_call(
        _tc1a_body,
        grid=(NB,),
        in_specs=[row1_spec, row1_spec,
                  w1_spec, b1_spec, w1_spec, b1_spec, b1_spec, b1_spec,
                  w1_spec, b1_spec, w1_spec, b1_spec, b1_spec, b1_spec],
        out_specs=[row1_spec, row1_spec],
        out_shape=[
            jax.ShapeDtypeStruct((N, D), jnp.float32),
            jax.ShapeDtypeStruct((N, D), jnp.float32),
        ],
    )(xt, x0,
      W1r, b1r.reshape(1, D), W2r, b2r.reshape(1, D),
      lnr_w.reshape(1, D), lnr_b.reshape(1, D),
      W1g, b1g.reshape(1, D), W2g, b2g.reshape(1, D),
      lng_w.reshape(1, D), lng_b.reshape(1, D))

    T, mfac = pl.pallas_call(
        _tc1b_body,
        grid=(NB, 2),
        in_specs=[
            pl.BlockSpec((_RBLK, D), lambda i, h: (i, 0)),
            pl.BlockSpec((_RBLK, 1), lambda i, h: (i, 0)),
            pl.BlockSpec((_NC, _RBLK, 1), lambda i, h: (0, i, 0)),
            pl.BlockSpec((_NC, _RBLK, 1), lambda i, h: (0, i, 0)),
            pl.BlockSpec((_RBLK, D), lambda i, h: (i, 0)),
        ],
        out_specs=[
            pl.BlockSpec((1, _RBLK, D), lambda i, h: (h, i, 0)),
            pl.BlockSpec((_RBLK, D), lambda i, h: (i, 0)),
        ],
        out_shape=[
            jax.ShapeDtypeStruct((2, N, D), jnp.float32),
            jax.ShapeDtypeStruct((N, D), jnp.float32),
        ],
    )(xt, ind_bd, deg_p.reshape(_NC, NPAD, 1), bcnt_p.reshape(_NC, NPAD, 1),
      rate)

    zrows = jnp.zeros((80, D), jnp.float32)
    part = _make_sc_agg(N, E, D, NPAD)(
        ei_flat, bd_flat, T.reshape(2 * N, D), zrows)

    out = pl.pallas_call(
        _tc2_body,
        grid=(NB,),
        in_specs=[
            pl.BlockSpec((_NC, _RBLK, D), lambda i: (0, i, 0)),
            pl.BlockSpec((_RBLK, D), lambda i: (i, 0)),
            pl.BlockSpec((_RBLK, D), lambda i: (i, 0)),
            pl.BlockSpec((D, D), lambda i: (0, 0)),
            pl.BlockSpec((1, D), lambda i: (0, 0)),
            pl.BlockSpec((D, D), lambda i: (0, 0)),
            pl.BlockSpec((1, D), lambda i: (0, 0)),
        ],
        out_specs=pl.BlockSpec((_RBLK, D), lambda i: (i, 0)),
        out_shape=jax.ShapeDtypeStruct((N, D), jnp.float32),
    )(part, mfac, gamma, W1f, b1f.reshape(1, D), W2f, b2f.reshape(1, D))
    return out


# speedup vs baseline: 1.1532x; 1.1532x over previous
import functools

import jax
import jax.numpy as jnp
from jax import lax
from jax.experimental import pallas as pl
from jax.experimental.pallas import tpu as pltpu
from jax.experimental.pallas import tpu_sc as plsc

_NC = 2
_NS = 16
_L = 16

_K = 80
_NGB = 5
_RBLK = 2000


def _gelu(x):
    return 0.5 * x * (1.0 + lax.erf(x * 0.7071067811865476))


def _ln(x, w, b):
    mu = jnp.mean(x, axis=-1, keepdims=True)
    xc = x - mu
    var = jnp.mean(xc * xc, axis=-1, keepdims=True)
    return xc * lax.rsqrt(var + 1e-5) * w + b


def _mlp(x, W1, b1, W2, b2):
    h = _gelu(jnp.dot(x, W1, preferred_element_type=jnp.float32) + b1)
    return jnp.dot(h, W2, preferred_element_type=jnp.float32) + b2


_SB = 400
_CPB = _SB // _K


def _make_sc_counts(N, E, NPAD):
    NW = _NC * _NS
    EW = E // NW
    NB = EW // _SB
    ZB = NPAD // _NS
    mesh = plsc.VectorSubcoreMesh(core_axis_name="c", subcore_axis_name="s")

    @functools.partial(
        pl.kernel,
        out_type=[
            jax.ShapeDtypeStruct((_NC, NPAD), jnp.int32),
            jax.ShapeDtypeStruct((_NC, NPAD), jnp.float32),
        ],
        mesh=mesh,
        compiler_params=pltpu.CompilerParams(needs_layout_passes=False),
        scratch_types=[
            pltpu.VMEM((N,), jnp.float32),
            pltpu.VMEM((_SB,), jnp.int32),
            pltpu.VMEM((_SB,), jnp.int32),
            pltpu.VMEM((_SB,), jnp.int32),
            pltpu.VMEM((_SB,), jnp.int32),
            pltpu.VMEM((_K,), jnp.float32),
            pltpu.VMEM((_K,), jnp.float32),
            pltpu.VMEM((_K,), jnp.int32),
            pltpu.VMEM((_K,), jnp.int32),
            pltpu.VMEM((_K,), jnp.int32),
            pltpu.VMEM((ZB,), jnp.int32),
            pltpu.VMEM((ZB,), jnp.float32),
            pltpu.VMEM_SHARED((NPAD,), jnp.int32),
            pltpu.VMEM_SHARED((NPAD,), jnp.float32),
            pltpu.SemaphoreType.DMA,
            pltpu.SemaphoreType.DMA,
            pltpu.SemaphoreType.DMA,
            pltpu.SemaphoreType.DMA,
        ],
    )
    def sc_counts(ei_h, bd_h, zi_h, zf_h, deg_h, bcnt_h,
                  bd_v, src_a, src_b, dst_a, dst_b, val_a, val_b,
                  dc_a, dc_b, ones_v, zi_v, zf_v, dacc, bacc,
                  ssem_a, ssem_b, asem_a, asem_b):
        c = lax.axis_index("c")
        s = lax.axis_index("s")
        wid = s * _NC + c
        base = wid * EW
        st = [(src_a, dst_a, ssem_a), (src_b, dst_b, ssem_b)]
        ch = [(val_a, dc_a, asem_a), (val_b, dc_b, asem_b)]

        def fire_stage(bidx, p):
            eb = base + bidx * _SB
            pltpu.async_copy(ei_h.at[pl.ds(eb, _SB)], st[p][0], st[p][2])
            pltpu.async_copy(ei_h.at[pl.ds(E + eb, _SB)], st[p][1], st[p][2])

        def wait_stage(p):
            pltpu.make_async_copy(ei_h.at[pl.ds(0, _SB)], st[p][0],
                                  st[p][2]).wait()
            pltpu.make_async_copy(ei_h.at[pl.ds(0, _SB)], st[p][1],
                                  st[p][2]).wait()

        def chunk(p_st, j, p, first):
            sst, dstt, _ = st[p_st]
            val, dc, asem = ch[p]
            if not first:
                pltpu.make_async_copy(ones_v, dacc.at[dc], asem).wait()
                pltpu.make_async_copy(val, bacc.at[dc], asem).wait()
            for k in range(_K // _L):
                sl_in = pl.ds(j * _K + k * _L, _L)
                sl = pl.ds(k * _L, _L)
                val[sl] = plsc.load_gather(bd_v, [sst[sl_in]])
                dc[sl] = dstt[sl_in]
            pltpu.async_copy(ones_v, dacc.at[dc], asem, add=True)
            pltpu.async_copy(val, bacc.at[dc], asem, add=True)

        fire_stage(0, 0)
        pltpu.sync_copy(bd_h, bd_v)
        pltpu.sync_copy(zi_h, zi_v)
        pltpu.sync_copy(zf_h, zf_v)
        pltpu.sync_copy(zi_v, dacc.at[pl.ds(s * ZB, ZB)])
        pltpu.sync_copy(zf_v, bacc.at[pl.ds(s * ZB, ZB)])
        for j in range(_K // _L):
            ones_v[pl.ds(j * _L, _L)] = jnp.ones((_L,), jnp.int32)
        plsc.subcore_barrier()

        wait_stage(0)
        fire_stage(1, 1)
        for j in range(_CPB):
            chunk(0, j, j % 2, first=(j < 2))
        wait_stage(1)
        fire_stage(2, 0)
        for j in range(_CPB):
            chunk(1, j, (j + 1) % 2, first=False)

        def body(g, carry):
            wait_stage(0)
            fire_stage(2 * g + 1, 1)
            for j in range(_CPB):
                chunk(0, j, j % 2, first=False)
            wait_stage(1)
            fire_stage(2 * g + 2, 0)
            for j in range(_CPB):
                chunk(1, j, (j + 1) % 2, first=False)
            return carry

        lax.fori_loop(1, NB // 2, body, 0)

        wait_stage(0)
        for j in range(_CPB):
            chunk(0, j, j % 2, first=False)
        for p in range(2):
            val, dc, asem = ch[p]
            pltpu.make_async_copy(ones_v, dacc.at[dc], asem).wait()
            pltpu.make_async_copy(val, bacc.at[dc], asem).wait()

        plsc.subcore_barrier()
        pltpu.sync_copy(dacc.at[pl.ds(s * ZB, ZB)], deg_h.at[c, pl.ds(s * ZB, ZB)])
        pltpu.sync_copy(bacc.at[pl.ds(s * ZB, ZB)], bcnt_h.at[c, pl.ds(s * ZB, ZB)])

    return sc_counts


def _make_sc_agg(N, E, D, NPAD):
    NW = _NC * _NS
    EW = E // NW
    NB = EW // _SB
    ZR = NPAD // _NS
    RB = 80
    mesh = plsc.VectorSubcoreMesh(core_axis_name="c", subcore_axis_name="s")

    @functools.partial(
        pl.kernel,
        out_type=jax.ShapeDtypeStruct((_NC, NPAD, D), jnp.float32),
        mesh=mesh,
        compiler_params=pltpu.CompilerParams(needs_layout_passes=False),
        scratch_types=[
            pltpu.VMEM((N,), jnp.float32),
            pltpu.VMEM((_SB,), jnp.int32),
            pltpu.VMEM((_SB,), jnp.int32),
            pltpu.VMEM((_SB,), jnp.int32),
            pltpu.VMEM((_SB,), jnp.int32),
            pltpu.VMEM((_K,), jnp.int32),
            pltpu.VMEM((_K,), jnp.int32),
            pltpu.VMEM((_K,), jnp.int32),
            pltpu.VMEM((_K,), jnp.int32),
            pltpu.VMEM((_K, D), jnp.float32),
            pltpu.VMEM((_K, D), jnp.float32),
            pltpu.VMEM_SHARED((NPAD, D), jnp.float32),
            pltpu.SemaphoreType.DMA,
            pltpu.SemaphoreType.DMA,
            pltpu.SemaphoreType.DMA,
            pltpu.SemaphoreType.DMA,
            pltpu.SemaphoreType.DMA,
            pltpu.SemaphoreType.DMA,
        ],
    )
    def sc_agg(ei_h, bd_h, t_h, zrows_h, part_h,
               bd_v, src_a, src_b, dst_a, dst_b, ei_a, ei_b, dc_a, dc_b,
               rows_a, rows_b, acc, ssem_a, ssem_b, gsem_a, gsem_b,
               csem_a, csem_b):
        c = lax.axis_index("c")
        s = lax.axis_index("s")
        wid = s * _NC + c
        base = wid * EW
        st = [(src_a, dst_a, ssem_a), (src_b, dst_b, ssem_b)]
        ch = [(ei_a, dc_a, rows_a, gsem_a, csem_a),
              (ei_b, dc_b, rows_b, gsem_b, csem_b)]

        def fire_stage(bidx, p):
            eb = base + bidx * _SB
            pltpu.async_copy(ei_h.at[pl.ds(eb, _SB)], st[p][0], st[p][2])
            pltpu.async_copy(ei_h.at[pl.ds(E + eb, _SB)], st[p][1], st[p][2])

        def wait_stage(p):
            pltpu.make_async_copy(ei_h.at[pl.ds(0, _SB)], st[p][0],
                                  st[p][2]).wait()
            pltpu.make_async_copy(ei_h.at[pl.ds(0, _SB)], st[p][1],
                                  st[p][2]).wait()

        def chunk(p_st, j, p, first=False, first2=False):
            sst, dstt, _ = st[p_st]
            ei, dc, rows, gsem, csem = ch[p]
            if not (first or first2):
                pltpu.make_async_copy(rows, acc.at[dc], csem).wait()
            for k in range(_K // _L):
                sl_in = pl.ds(j * _K + k * _L, _L)
                sl = pl.ds(k * _L, _L)
                d16 = dstt[sl_in]
                bdd = plsc.load_gather(bd_v, [d16]).astype(jnp.int32)
                ei[sl] = sst[sl_in] + bdd * N
                dc[sl] = d16
            pltpu.async_copy(t_h.at[ei], rows, gsem)
            if not first:
                eo, do, ro, go, co = ch[1 - p]
                pltpu.make_async_copy(t_h.at[eo], ro, go).wait()
                pltpu.async_copy(ro, acc.at[do], co, add=True)

        def final_drain():
            eo, do, ro, go, co = ch[0]
            pltpu.make_async_copy(t_h.at[eo], ro, go).wait()
            pltpu.async_copy(ro, acc.at[do], co, add=True)
            pltpu.make_async_copy(ro, acc.at[do], co).wait()
            e1, d1, r1, g1, c1 = ch[1]
            pltpu.make_async_copy(r1, acc.at[d1], c1).wait()

        fire_stage(0, 0)
        pltpu.sync_copy(bd_h, bd_v)
        pltpu.sync_copy(zrows_h, rows_a)
        for t in range(ZR // RB):
            pltpu.sync_copy(rows_a, acc.at[pl.ds(s * ZR + t * RB, RB)])
        plsc.subcore_barrier()

        wait_stage(0)
        fire_stage(1, 1)
        for j in range(_CPB):
            chunk(0, j, j % 2, first=(j == 0), first2=(j == 1))
        wait_stage(1)
        fire_stage(2, 0)
        for j in range(_CPB):
            chunk(1, j, (j + 1) % 2, first=False)

        def body(g, carry):
            wait_stage(0)
            fire_stage(2 * g + 1, 1)
            for j in range(_CPB):
                chunk(0, j, j % 2, first=False)
            wait_stage(1)
            fire_stage(2 * g + 2, 0)
            for j in range(_CPB):
                chunk(1, j, (j + 1) % 2, first=False)
            return carry

        lax.fori_loop(1, NB // 2, body, 0)

        wait_stage(0)
        for j in range(_CPB):
            chunk(0, j, j % 2)
        final_drain()

        plsc.subcore_barrier()
        for t in range(ZR // RB):
            r0 = s * ZR + t * RB
            pltpu.sync_copy(acc.at[pl.ds(r0, RB)], part_h.at[c, pl.ds(r0, RB)])

    return sc_agg


def _tc1a_body(xt_ref, x0_ref,
               W1r_ref, b1r_ref, W2r_ref, b2r_ref, lnrw_ref, lnrb_ref,
               W1g_ref, b1g_ref, W2g_ref, b2g_ref, lngw_ref, lngb_ref,
               rate_ref, gamma_ref):
    rate_ref[...] = _ln(_mlp(xt_ref[...], W1r_ref[...], b1r_ref[...],
                             W2r_ref[...], b2r_ref[...]),
                        lnrw_ref[...], lnrb_ref[...])
    gamma_ref[...] = _ln(_mlp(x0_ref[...], W1g_ref[...], b1g_ref[...],
                              W2g_ref[...], b2g_ref[...]),
                         lngw_ref[...], lngb_ref[...])


def _tc1b_body(xt_ref, bd_ref, deg_ref, bc_ref, rate_ref,
               T_ref, mfac_ref):
    h = pl.program_id(1)
    bd = bd_ref[...]
    deg = (deg_ref[0] + deg_ref[1]).astype(jnp.float32)
    bc = bc_ref[0] + bc_ref[1]
    p = 1.0 + bd * bc + (1.0 - bd) * (deg - bc)
    rinv = lax.rsqrt(p)
    xt = xt_ref[...]

    @pl.when(h == 0)
    def _():
        mfac_ref[...] = ((1.0 - bd) + bd * rate_ref[...]) * rinv
        T_ref[0] = (2.0 - bd) * rinv * xt

    @pl.when(h == 1)
    def _():
        T_ref[0] = (1.0 - bd) * rinv * xt


def _tc2_body(part_ref, mfac_ref, gamma_ref, W1f_ref, b1f_ref, W2f_ref,
              b2f_ref, out_ref):
    agg = part_ref[0] + part_ref[1]
    z = mfac_ref[...] * agg
    out_ref[...] = _mlp(z, W1f_ref[...], b1f_ref[...], W2f_ref[...],
                        b2f_ref[...]) + gamma_ref[...]


def kernel(xt, x0, edge_index, ind_bd, W1r, b1r, W2r, b2r, lnr_w, lnr_b,
           W1g, b1g, W2g, b2g, lng_w, lng_b, W1f, b1f, W2f, b2f):
    N, D = xt.shape
    E = edge_index.shape[1]
    NPAD = -(-N // 1280) * 1280
    NB = N // _RBLK

    bd_flat = ind_bd.reshape(N)

    zeros_i = jnp.zeros((NPAD // _NS,), jnp.int32)
    zeros_f = jnp.zeros((NPAD // _NS,), jnp.float32)
    ei_flat = edge_index.reshape(2 * E)
    deg_p, bcnt_p = _make_sc_counts(N, E, NPAD)(
        ei_flat, bd_flat, zeros_i, zeros_f)

    row1_spec = pl.BlockSpec((_RBLK, D), lambda i: (i, 0))
    w1_spec = pl.BlockSpec((D, D), lambda i: (0, 0))
    b1_spec = pl.BlockSpec((1, D), lambda i: (0, 0))
    rate, gamma = pl.pallas_call(
        _tc1a_body,
        grid=(NB,),
        in_specs=[row1_spec, row1_spec,
                  w1_spec, b1_spec, w1_spec, b1_spec, b1_spec, b1_spec,
                  w1_spec, b1_spec, w1_spec, b1_spec, b1_spec, b1_spec],
        out_specs=[row1_spec, row1_spec],
        out_shape=[
            jax.ShapeDtypeStruct((N, D), jnp.float32),
            jax.ShapeDtypeStruct((N, D), jnp.float32),
        ],
    )(xt, x0,
      W1r, b1r.reshape(1, D), W2r, b2r.reshape(1, D),
      lnr_w.reshape(1, D), lnr_b.reshape(1, D),
      W1g, b1g.reshape(1, D), W2g, b2g.reshape(1, D),
      lng_w.reshape(1, D), lng_b.reshape(1, D))

    T, mfac = pl.pallas_call(
        _tc1b_body,
        grid=(NB, 2),
        in_specs=[
            pl.BlockSpec((_RBLK, D), lambda i, h: (i, 0)),
            pl.BlockSpec((_RBLK, 1), lambda i, h: (i, 0)),
            pl.BlockSpec((_NC, _RBLK, 1), lambda i, h: (0, i, 0)),
            pl.BlockSpec((_NC, _RBLK, 1), lambda i, h: (0, i, 0)),
            pl.BlockSpec((_RBLK, D), lambda i, h: (i, 0)),
        ],
        out_specs=[
            pl.BlockSpec((1, _RBLK, D), lambda i, h: (h, i, 0)),
            pl.BlockSpec((_RBLK, D), lambda i, h: (i, 0)),
        ],
        out_shape=[
            jax.ShapeDtypeStruct((2, N, D), jnp.float32),
            jax.ShapeDtypeStruct((N, D), jnp.float32),
        ],
    )(xt, ind_bd, deg_p.reshape(_NC, NPAD, 1), bcnt_p.reshape(_NC, NPAD, 1),
      rate)

    zrows = jnp.zeros((80, D), jnp.float32)
    part = _make_sc_agg(N, E, D, NPAD)(
        ei_flat, bd_flat, T.reshape(2 * N, D), zrows)

    out = pl.pallas_call(
        _tc2_body,
        grid=(NB,),
        in_specs=[
            pl.BlockSpec((_NC, _RBLK, D), lambda i: (0, i, 0)),
            pl.BlockSpec((_RBLK, D), lambda i: (i, 0)),
            pl.BlockSpec((_RBLK, D), lambda i: (i, 0)),
            pl.BlockSpec((D, D), lambda i: (0, 0)),
            pl.BlockSpec((1, D), lambda i: (0, 0)),
            pl.BlockSpec((D, D), lambda i: (0, 0)),
            pl.BlockSpec((1, D), lambda i: (0, 0)),
        ],
        out_specs=pl.BlockSpec((_RBLK, D), lambda i: (i, 0)),
        out_shape=jax.ShapeDtypeStruct((N, D), jnp.float32),
    )(part, mfac, gamma, W1f, b1f.reshape(1, D), W2f, b2f.reshape(1, D))
    return out
